# baseline (speedup 1.0000x reference)
import functools

import jax
import jax.numpy as jnp
from jax import lax
from jax.experimental import pallas as pl
from jax.experimental.pallas import tpu as pltpu
from jax.experimental.pallas import tpu_sc as plsc

N = 8192
E = 8
D = 512
DH = D // 2
ROW = E + E * D

NC = 2
NS = 16
L = 16
NW = NC * NS
TOK_W = N // NW
CT = 32
NCHUNK = TOK_W // CT

SEL_BLK = 512
SEL_GRID = N // SEL_BLK


def _selector_body(lg_ref, u_ref, off_ref):
    b = pl.program_id(0)
    u = u_ref[...]
    uc = jnp.clip(u, 1e-6, 1.0 - 1e-6)
    g = -jnp.log(-jnp.log(uc))
    s = lg_ref[...] + g
    m = jnp.max(s, axis=0, keepdims=True)
    idx = lax.broadcasted_iota(jnp.int32, (E, SEL_BLK), 0)
    sel = jnp.min(jnp.where(s == m, idx, E), axis=0, keepdims=True)
    tok = b * SEL_BLK + lax.broadcasted_iota(jnp.int32, (1, SEL_BLK), 1)
    off_ref[0] = tok * ROW + E + sel * D


def _compute_offsets(logits_t, u_t):
    out = pl.pallas_call(
        _selector_body,
        grid=(SEL_GRID,),
        in_specs=[
            pl.BlockSpec((E, SEL_BLK), lambda b: (0, b)),
            pl.BlockSpec((E, SEL_BLK), lambda b: (0, b)),
        ],
        out_specs=pl.BlockSpec((1, 1, SEL_BLK), lambda b: (b, 0, 0)),
        out_shape=jax.ShapeDtypeStruct((SEL_GRID, 1, SEL_BLK), jnp.int32),
    )(logits_t, u_t)
    return out.reshape(N)


def _gather_combine_body(flat_hbm, off_hbm, eps_hbm, out_hbm,
                         off_v, rows_v, eps_v, out_v, sem_g, sem_e):
    c = lax.axis_index("c")
    s = lax.axis_index("s")
    wid = s * NC + c
    t0 = wid * TOK_W
    pltpu.sync_copy(off_hbm.at[pl.ds(pl.multiple_of(t0, 8), TOK_W)], off_v)

    def chunk(ci, carry):
        e0 = pl.multiple_of((t0 + ci * CT) * DH, 8)
        eps_cp = pltpu.async_copy(eps_hbm.at[pl.ds(e0, CT * DH)], eps_v, sem_e)
        cps = []
        for gi in range(CT // L):
            offv = off_v[pl.ds(pl.multiple_of(ci * CT + gi * L, L), L)]
            for j in range(L):
                t = gi * L + j
                off = pl.multiple_of(offv[j], 8)
                cps.append(pltpu.async_copy(
                    flat_hbm.at[pl.ds(off, D)],
                    rows_v.at[pl.ds(t * D, D)], sem_g))
        eps_cp.wait()
        for cp in cps:
            cp.wait()

        def vec(v, carry2):
            t = v // (DH // L)
            k = (v % (DH // L)) * L
            mb = pl.multiple_of(t * D + k, L)
            eb = pl.multiple_of(v * L, L)
            mean = rows_v[pl.ds(mb, L)]
            lsd = rows_v[pl.ds(mb + DH, L)]
            ep = eps_v[pl.ds(eb, L)]
            out_v[pl.ds(eb, L)] = mean + ep * jnp.exp(lsd)
            return carry2

        lax.fori_loop(0, CT * (DH // L), vec, 0, unroll=4)
        pltpu.sync_copy(out_v, out_hbm.at[pl.ds(e0, CT * DH)])
        return carry

    lax.fori_loop(0, NCHUNK, chunk, 0)


@functools.lru_cache(maxsize=1)
def _build_gather_combine():
    return pl.kernel(
        _gather_combine_body,
        out_type=jax.ShapeDtypeStruct((N * DH,), jnp.float32),
        mesh=plsc.VectorSubcoreMesh(
            core_axis_name="c", subcore_axis_name="s",
            num_cores=NC, num_subcores=NS),
        scratch_types=[
            pltpu.VMEM((TOK_W,), jnp.int32),
            pltpu.VMEM((CT * D,), jnp.float32),
            pltpu.VMEM((CT * DH,), jnp.float32),
            pltpu.VMEM((CT * DH,), jnp.float32),
            pltpu.SemaphoreType.DMA,
            pltpu.SemaphoreType.DMA,
        ],
    )


def kernel(params, u, eps):
    logits_t = params[:, :E].T
    u_t = u.T
    offs = _compute_offsets(logits_t, u_t)
    flat = params.reshape(-1)
    out = _build_gather_combine()(flat, offs, eps.reshape(-1))
    return out.reshape(N, DH)


# speedup vs baseline: 1.0519x; 1.0519x over previous
import functools

import jax
import jax.numpy as jnp
from jax import lax
from jax.experimental import pallas as pl
from jax.experimental.pallas import tpu as pltpu
from jax.experimental.pallas import tpu_sc as plsc

N = 8192
E = 8
D = 512
DH = D // 2
ROW = E + E * D

NC = 2
NS = 16
L = 16
NW = NC * NS
TOK_W = N // NW
CT = 32
NCHUNK = TOK_W // CT

SEL_BLK = 512
SEL_GRID = N // SEL_BLK


def _selector_body(lg_ref, u_ref, col_ref):
    u = u_ref[...]
    uc = jnp.clip(u, 1e-6, 1.0 - 1e-6)
    g = -jnp.log(-jnp.log(uc))
    s = lg_ref[...] + g
    m = jnp.max(s, axis=0, keepdims=True)
    idx = lax.broadcasted_iota(jnp.int32, (E, SEL_BLK), 0)
    sel = jnp.min(jnp.where(s == m, idx, E), axis=0)
    col_ref[...] = E + sel * D


def _compute_cols(logits_t, u_t):
    return pl.pallas_call(
        _selector_body,
        grid=(SEL_GRID,),
        in_specs=[
            pl.BlockSpec((E, SEL_BLK), lambda b: (0, b)),
            pl.BlockSpec((E, SEL_BLK), lambda b: (0, b)),
        ],
        out_specs=pl.BlockSpec((SEL_BLK,), lambda b: (b,)),
        out_shape=jax.ShapeDtypeStruct((N,), jnp.int32),
    )(logits_t, u_t)


NG = TOK_W // 8


def _gather_combine_body(params_hbm, col_hbm, eps_hbm, out_hbm,
                         col_v, pblk_v, eps_v, out_v, sem_p, sem_e):
    c = lax.axis_index("c")
    s = lax.axis_index("s")
    wid = s * NC + c
    t0 = wid * TOK_W
    pltpu.sync_copy(col_hbm.at[pl.ds(pl.multiple_of(t0, 8), TOK_W)],
                    col_v.at[pl.ds(0, TOK_W)])
    iota = lax.iota(jnp.int32, L)

    def grp(g, carry):
        r0 = pl.multiple_of(t0 + g * 8, 8)
        pc = pltpu.async_copy(params_hbm.at[pl.ds(r0, 8), :], pblk_v, sem_p)
        ec = pltpu.async_copy(eps_hbm.at[pl.ds(r0, 8), :], eps_v, sem_e)
        colv = col_v[pl.ds(pl.multiple_of(g * 8, 8), L)]
        pc.wait()
        ec.wait()
        for j in range(8):
            cb = colv[j]
            jvec = jnp.full((L,), j, jnp.int32)
            for m in range(DH // L):
                idxm = cb + (m * L) + iota
                mean = plsc.load_gather(pblk_v, [jvec, idxm])
                lsd = plsc.load_gather(pblk_v, [jvec, idxm + DH])
                ep = eps_v[j, pl.ds(m * L, L)]
                out_v[j, pl.ds(m * L, L)] = mean + ep * jnp.exp(lsd)
        pltpu.sync_copy(out_v, out_hbm.at[pl.ds(r0, 8), :])
        return carry

    lax.fori_loop(0, NG, grp, 0)


@functools.lru_cache(maxsize=1)
def _build_gather_combine():
    return pl.kernel(
        _gather_combine_body,
        out_type=jax.ShapeDtypeStruct((N, DH), jnp.float32),
        mesh=plsc.VectorSubcoreMesh(
            core_axis_name="c", subcore_axis_name="s",
            num_cores=NC, num_subcores=NS),
        compiler_params=pltpu.CompilerParams(
            use_tc_tiling_on_sc=True, needs_layout_passes=False),
        scratch_types=[
            pltpu.VMEM((TOK_W + L,), jnp.int32),
            pltpu.VMEM((8, ROW), jnp.float32),
            pltpu.VMEM((8, DH), jnp.float32),
            pltpu.VMEM((8, DH), jnp.float32),
            pltpu.SemaphoreType.DMA,
            pltpu.SemaphoreType.DMA,
        ],
    )


def kernel(params, u, eps):
    logits_t = params[:, :E].T
    u_t = u.T
    cols = _compute_cols(logits_t, u_t)
    return _build_gather_combine()(params, cols, eps)


# speedup vs baseline: 1.2411x; 1.1799x over previous
import functools

import jax
import jax.numpy as jnp
from jax import lax
from jax.experimental import pallas as pl
from jax.experimental.pallas import tpu as pltpu
from jax.experimental.pallas import tpu_sc as plsc

N = 8192
E = 8
D = 512
DH = D // 2
ROW = E + E * D

NC = 2
NS = 16
L = 16
NW = NC * NS
TOK_W = N // NW
CT = 32
NCHUNK = TOK_W // CT

SEL_BLK = 512
SEL_GRID = N // SEL_BLK


def _selector_body(lg_ref, u_ref, col_ref):
    u = u_ref[...]
    uc = jnp.clip(u, 1e-6, 1.0 - 1e-6)
    g = -jnp.log(-jnp.log(uc))
    s = lg_ref[...] + g
    m = jnp.max(s, axis=0, keepdims=True)
    idx = lax.broadcasted_iota(jnp.int32, (E, SEL_BLK), 0)
    sel = jnp.min(jnp.where(s == m, idx, E), axis=0)
    col_ref[...] = E + sel * D


def _compute_cols(logits_t, u_t):
    return pl.pallas_call(
        _selector_body,
        grid=(SEL_GRID,),
        in_specs=[
            pl.BlockSpec((E, SEL_BLK), lambda b: (0, b)),
            pl.BlockSpec((E, SEL_BLK), lambda b: (0, b)),
        ],
        out_specs=pl.BlockSpec((SEL_BLK,), lambda b: (b,)),
        out_shape=jax.ShapeDtypeStruct((N,), jnp.int32),
    )(logits_t, u_t)


NG = TOK_W // 8


def _gather_combine_body(params_hbm, col_hbm, eps_hbm, out_hbm,
                         col_v, pblk_v, eps_v, out_v,
                         sem_p0, sem_p1, sem_e0, sem_e1, sem_o0, sem_o1):
    sem_p = (sem_p0, sem_p1)
    sem_e = (sem_e0, sem_e1)
    sem_o = (sem_o0, sem_o1)
    c = lax.axis_index("c")
    s = lax.axis_index("s")
    wid = s * NC + c
    t0 = wid * TOK_W
    pltpu.sync_copy(col_hbm.at[pl.ds(pl.multiple_of(t0, 8), TOK_W)],
                    col_v.at[pl.ds(0, TOK_W)])
    iota = lax.iota(jnp.int32, L)

    def rows(g):
        return pl.ds(pl.multiple_of(t0 + g * 8, 8), 8)

    def issue(g, b):
        pltpu.async_copy(params_hbm.at[rows(g), :], pblk_v.at[b], sem_p[b])
        pltpu.async_copy(eps_hbm.at[rows(g), :], eps_v.at[b], sem_e[b])

    issue(0, 0)

    def half(g, b):
        gn = g + 1

        @pl.when(gn < NG)
        def _():
            issue(gn, 1 - b)

        pltpu.make_async_copy(
            params_hbm.at[rows(g), :], pblk_v.at[b], sem_p[b]).wait()
        pltpu.make_async_copy(
            eps_hbm.at[rows(g), :], eps_v.at[b], sem_e[b]).wait()

        @pl.when(g >= 2)
        def _():
            pltpu.make_async_copy(
                out_v.at[b], out_hbm.at[rows(g - 2), :], sem_o[b]).wait()

        colv = col_v[pl.ds(pl.multiple_of(g * 8, 8), L)]
        for j in range(8):
            cb = colv[j]
            jvec = jnp.full((L,), j, jnp.int32)
            for m in range(DH // L):
                idxm = cb + (m * L) + iota
                mean = plsc.load_gather(pblk_v, [jnp.full((L,), b, jnp.int32),
                                                 jvec, idxm])
                lsd = plsc.load_gather(pblk_v, [jnp.full((L,), b, jnp.int32),
                                                jvec, idxm + DH])
                ep = eps_v[b, j, pl.ds(m * L, L)]
                out_v[b, j, pl.ds(m * L, L)] = mean + ep * jnp.exp(lsd)
        pltpu.async_copy(out_v.at[b], out_hbm.at[rows(g), :], sem_o[b])

    def grp2(g2, carry):
        half(2 * g2, 0)
        half(2 * g2 + 1, 1)
        return carry

    lax.fori_loop(0, NG // 2, grp2, 0)
    pltpu.make_async_copy(
        out_v.at[0], out_hbm.at[rows(NG - 2), :], sem_o[0]).wait()
    pltpu.make_async_copy(
        out_v.at[1], out_hbm.at[rows(NG - 1), :], sem_o[1]).wait()


@functools.lru_cache(maxsize=1)
def _build_gather_combine():
    return pl.kernel(
        _gather_combine_body,
        out_type=jax.ShapeDtypeStruct((N, DH), jnp.float32),
        mesh=plsc.VectorSubcoreMesh(
            core_axis_name="c", subcore_axis_name="s",
            num_cores=NC, num_subcores=NS),
        compiler_params=pltpu.CompilerParams(
            use_tc_tiling_on_sc=True, needs_layout_passes=False),
        scratch_types=[
            pltpu.VMEM((TOK_W + L,), jnp.int32),
            pltpu.VMEM((2, 8, ROW), jnp.float32),
            pltpu.VMEM((2, 8, DH), jnp.float32),
            pltpu.VMEM((2, 8, DH), jnp.float32),
            pltpu.SemaphoreType.DMA,
            pltpu.SemaphoreType.DMA,
            pltpu.SemaphoreType.DMA,
            pltpu.SemaphoreType.DMA,
            pltpu.SemaphoreType.DMA,
            pltpu.SemaphoreType.DMA,
        ],
    )


def kernel(params, u, eps):
    logits_t = params[:, :E].T
    u_t = u.T
    cols = _compute_cols(logits_t, u_t)
    return _build_gather_combine()(params, cols, eps)


# speedup vs baseline: 3.1899x; 2.5702x over previous
import jax
import jax.numpy as jnp
from jax import lax
from jax.experimental import pallas as pl

N = 8192
E = 8
D = 512
DH = D // 2
ROW = E + E * D

TB = 128
GRID = N // TB


def _fused_body(pT_ref, uT_ref, epsT_ref, outT_ref):
    u = uT_ref[...]
    uc = jnp.clip(u, 1e-6, 1.0 - 1e-6)
    g = -jnp.log(-jnp.log(uc))
    s = pT_ref[0:E, :] + g
    m = jnp.max(s, axis=0, keepdims=True)
    idx = lax.broadcasted_iota(jnp.int32, (E, TB), 0)
    sel = jnp.min(jnp.where(s == m, idx, E), axis=0, keepdims=True)

    mean = pT_ref[E:E + DH, :]
    lsd = pT_ref[E + DH:E + D, :]
    for e in range(1, E):
        msk = sel == e
        mean = jnp.where(msk, pT_ref[E + e * D:E + e * D + DH, :], mean)
        lsd = jnp.where(msk, pT_ref[E + e * D + DH:E + (e + 1) * D, :], lsd)
    outT_ref[...] = mean + epsT_ref[...] * jnp.exp(lsd)


def kernel(params, u, eps):
    pT = params.T
    uT = u.T
    epsT = eps.T
    outT = pl.pallas_call(
        _fused_body,
        grid=(GRID,),
        in_specs=[
            pl.BlockSpec((ROW, TB), lambda b: (0, b)),
            pl.BlockSpec((E, TB), lambda b: (0, b)),
            pl.BlockSpec((DH, TB), lambda b: (0, b)),
        ],
        out_specs=pl.BlockSpec((DH, TB), lambda b: (0, b)),
        out_shape=jax.ShapeDtypeStruct((DH, N), jnp.float32),
    )(pT, uT, epsT)
    return outT.T


# speedup vs baseline: 4.8069x; 1.5069x over previous
import jax
import jax.numpy as jnp
from jax import lax
from jax.experimental import pallas as pl

N = 8192
E = 8
D = 512
DH = D // 2
ROW = E + E * D

TB = 128
GRID = N // TB


def _fused_body(pT_ref, uT_ref, eps_ref, out_ref):
    u = uT_ref[...]
    uc = jnp.clip(u, 1e-6, 1.0 - 1e-6)
    g = -jnp.log(-jnp.log(uc))
    s = pT_ref[0:E, :] + g
    m = jnp.max(s, axis=0, keepdims=True)
    idx = lax.broadcasted_iota(jnp.int32, (E, TB), 0)
    sel = jnp.min(jnp.where(s == m, idx, E), axis=0, keepdims=True)

    mean = pT_ref[E:E + DH, :]
    lsd = pT_ref[E + DH:E + D, :]
    for e in range(1, E):
        msk = sel == e
        mean = jnp.where(msk, pT_ref[E + e * D:E + e * D + DH, :], mean)
        lsd = jnp.where(msk, pT_ref[E + e * D + DH:E + (e + 1) * D, :], lsd)
    mean_t = jnp.transpose(mean, (1, 0))
    lsd_t = jnp.transpose(lsd, (1, 0))
    out_ref[...] = mean_t + eps_ref[...] * jnp.exp(lsd_t)


def kernel(params, u, eps):
    pT = params.T
    uT = u.T
    return pl.pallas_call(
        _fused_body,
        grid=(GRID,),
        in_specs=[
            pl.BlockSpec((ROW, TB), lambda b: (0, b)),
            pl.BlockSpec((E, TB), lambda b: (0, b)),
            pl.BlockSpec((TB, DH), lambda b: (b, 0)),
        ],
        out_specs=pl.BlockSpec((TB, DH), lambda b: (b, 0)),
        out_shape=jax.ShapeDtypeStruct((N, DH), jnp.float32),
    )(pT, uT, eps)
